# baseline (speedup 1.0000x reference)
import functools

import jax
import jax.numpy as jnp
from jax.experimental import pallas as pl
from jax.experimental.pallas import tpu as pltpu

EMB = 256
IN = 2 * EMB


def _body(tm, adj_ref, a_ref, h0_ref, wagg_ref, bagg_ref, wfcra_ref,
          bfcra_ref, out_ref, rhs_ref, h_ref):
    k = pl.program_id(0)
    r = pl.program_id(1)
    depth = pl.num_programs(0)
    n = h_ref.shape[0]

    @pl.when(jnp.logical_and(k == 0, r == 0))
    def _():
        h_ref[...] = h0_ref[...]

    @pl.when(r == 0)
    def _():
        rhs_ref[:, :IN] = a_ref[0].astype(jnp.bfloat16)
        rhs_ref[:, IN:] = jnp.ones((n, 128), jnp.bfloat16)

    row = pl.ds(pl.multiple_of(r * tm, tm), tm)
    adj_blk = adj_ref[0].astype(jnp.bfloat16)
    acc = jnp.dot(adj_blk, rhs_ref[...], preferred_element_type=jnp.float32)
    deg_inv = 1.0 / jnp.maximum(acc[:, IN:IN + 1], 1e-6)
    agg = (acc[:, :IN] * deg_inv).astype(jnp.bfloat16)
    wagg = wagg_ref[0].astype(jnp.bfloat16)
    emb = jnp.dot(agg, wagg, preferred_element_type=jnp.float32)
    emb = jnp.maximum(emb + bagg_ref[0], 0.0).astype(jnp.bfloat16)
    wf = wfcra_ref[0].astype(jnp.bfloat16)
    h_prev = h_ref[row, :].astype(jnp.bfloat16)
    h_new = (jnp.dot(emb, wf[:EMB], preferred_element_type=jnp.float32)
             + jnp.dot(h_prev, wf[EMB:], preferred_element_type=jnp.float32)
             + bfcra_ref[0])
    h_new = jnp.maximum(h_new, 0.0)
    h_ref[row, :] = h_new

    @pl.when(k == depth - 1)
    def _():
        out_ref[...] = h_new


def kernel(h0, a, adjacent_mat, W_agg, b_agg, W_fcra, b_fcra):
    n = h0.shape[0]
    depth = a.shape[0]
    nr = 4
    tm = n // nr
    b_agg3 = b_agg.reshape(depth, 1, EMB)
    b_fcra3 = b_fcra.reshape(depth, 1, EMB)

    grid = (depth, nr)
    out = pl.pallas_call(
        functools.partial(_body, tm),
        grid=grid,
        in_specs=[
            pl.BlockSpec((1, tm, n), lambda k, r: (k, r, 0)),
            pl.BlockSpec((1, n, IN), lambda k, r: (k, 0, 0)),
            pl.BlockSpec((n, EMB), lambda k, r: (0, 0)),
            pl.BlockSpec((1, IN, EMB), lambda k, r: (k, 0, 0)),
            pl.BlockSpec((1, 1, EMB), lambda k, r: (k, 0, 0)),
            pl.BlockSpec((1, IN, EMB), lambda k, r: (k, 0, 0)),
            pl.BlockSpec((1, 1, EMB), lambda k, r: (k, 0, 0)),
        ],
        out_specs=pl.BlockSpec((tm, EMB), lambda k, r: (r, 0)),
        out_shape=jax.ShapeDtypeStruct((n, EMB), jnp.float32),
        scratch_shapes=[
            pltpu.VMEM((n, IN + 128), jnp.bfloat16),
            pltpu.VMEM((n, EMB), jnp.float32),
        ],
        compiler_params=pltpu.CompilerParams(
            dimension_semantics=("arbitrary", "arbitrary"),
        ),
    )(adjacent_mat, a, h0, W_agg, b_agg3, W_fcra, b_fcra3)
    return out


# speedup vs baseline: 1.3407x; 1.3407x over previous
import functools

import jax
import jax.numpy as jnp
from jax.experimental import pallas as pl
from jax.experimental.pallas import tpu as pltpu

EMB = 256
IN = 2 * EMB


def _body(adj_ref, a_ref, h0_ref, wagg_ref, bagg_ref, wfcra_ref,
          bfcra_ref, out_ref, h_ref):
    k = pl.program_id(0)
    depth = pl.num_programs(0)

    @pl.when(k == 0)
    def _():
        h_ref[...] = h0_ref[...]

    adj_blk = adj_ref[0]
    acc = jnp.dot(adj_blk.astype(jnp.bfloat16), a_ref[0].astype(jnp.bfloat16),
                  preferred_element_type=jnp.float32)
    deg = jnp.sum(adj_blk, axis=1, keepdims=True)
    deg_inv = 1.0 / jnp.maximum(deg, 1e-6)
    agg = (acc * deg_inv).astype(jnp.bfloat16)
    wagg = wagg_ref[0].astype(jnp.bfloat16)
    emb = jnp.dot(agg, wagg, preferred_element_type=jnp.float32)
    emb = jnp.maximum(emb + bagg_ref[0], 0.0).astype(jnp.bfloat16)
    wf = wfcra_ref[0].astype(jnp.bfloat16)
    h_prev = h_ref[...].astype(jnp.bfloat16)
    h_new = (jnp.dot(emb, wf[:EMB], preferred_element_type=jnp.float32)
             + jnp.dot(h_prev, wf[EMB:], preferred_element_type=jnp.float32)
             + bfcra_ref[0])
    h_new = jnp.maximum(h_new, 0.0)
    h_ref[...] = h_new

    @pl.when(k == depth - 1)
    def _():
        out_ref[...] = h_new


def kernel(h0, a, adjacent_mat, W_agg, b_agg, W_fcra, b_fcra):
    n = h0.shape[0]
    depth = a.shape[0]
    b_agg3 = b_agg.reshape(depth, 1, EMB)
    b_fcra3 = b_fcra.reshape(depth, 1, EMB)

    grid = (depth,)
    out = pl.pallas_call(
        _body,
        grid=grid,
        in_specs=[
            pl.BlockSpec((1, n, n), lambda k: (k, 0, 0)),
            pl.BlockSpec((1, n, IN), lambda k: (k, 0, 0)),
            pl.BlockSpec((n, EMB), lambda k: (0, 0)),
            pl.BlockSpec((1, IN, EMB), lambda k: (k, 0, 0)),
            pl.BlockSpec((1, 1, EMB), lambda k: (k, 0, 0)),
            pl.BlockSpec((1, IN, EMB), lambda k: (k, 0, 0)),
            pl.BlockSpec((1, 1, EMB), lambda k: (k, 0, 0)),
        ],
        out_specs=pl.BlockSpec((n, EMB), lambda k: (0, 0)),
        out_shape=jax.ShapeDtypeStruct((n, EMB), jnp.float32),
        scratch_shapes=[
            pltpu.VMEM((n, EMB), jnp.float32),
        ],
        compiler_params=pltpu.CompilerParams(
            dimension_semantics=("arbitrary",),
        ),
    )(adjacent_mat, a, h0, W_agg, b_agg3, W_fcra, b_fcra3)
    return out
